# baseline (speedup 1.0000x reference)
import functools

import jax
import jax.numpy as jnp
from jax import lax
from jax.experimental import pallas as pl
from jax.experimental.pallas import tpu as pltpu
from jax.experimental.pallas import tpu_sc as plsc

LOW_THRESH = 0.3
HIGH_THRESH = 0.7

_NW = 32
_LANES = 16
_B = 2560


def _matcher_body(an_ref, gt_ref, out_ref, q_ref, gm_ref, *, n, gp, g, nb):
    gx1 = gt_ref[:, 0:1]
    gy1 = gt_ref[:, 1:2]
    gx2 = gt_ref[:, 2:3]
    gy2 = gt_ref[:, 3:4]
    ga = (gx2 - gx1) * (gy2 - gy1)
    widths = [min(_B, n - j * _B) for j in range(nb)]
    for j in range(nb):
        w_ = widths[j]
        at = jnp.transpose(an_ref[pl.ds(j * _B, w_), :])
        ax1 = at[0:1, :]
        ay1 = at[1:2, :]
        ax2 = at[2:3, :]
        ay2 = at[3:4, :]
        ab = (ax2 - ax1) * (ay2 - ay1)
        w = jnp.maximum(jnp.minimum(gx2, ax2) - jnp.maximum(gx1, ax1), 0.0)
        h = jnp.maximum(jnp.minimum(gy2, ay2) - jnp.maximum(gy1, ay1), 0.0)
        inter = w * h
        q = inter / (ga + ab - inter)
        q_ref[j, :, 0:w_] = q
        bm = jnp.max(q, axis=1, keepdims=True)
        if j == 0:
            gm_ref[:, 0:1] = bm
        else:
            gm_ref[:, 0:1] = jnp.maximum(gm_ref[:, 0:1], bm)
    gm = gm_ref[:, 0:1]
    for j in range(nb):
        w_ = widths[j]
        q = q_ref[j, :, 0:w_]
        giota = lax.broadcasted_iota(jnp.int32, (gp, w_), 0)
        mv = jnp.max(q, axis=0, keepdims=True)
        am = jnp.min(jnp.where(q == mv, giota, gp), axis=0, keepdims=True)
        restore = jnp.any((q == gm) & (giota < g), axis=0, keepdims=True)
        m = jnp.where(mv < LOW_THRESH, -1, jnp.where(mv < HIGH_THRESH, -2, am))
        m = jnp.where(restore, am, m)
        out_ref[j:j + 1, 0:w_] = m


def _sc_labels_body(m_hbm, gt_hbm, s_hbm, c_hbm, out_hbm,
                    m_v, tbl_v, s_v, c_v, o_v, *, n, g, nc):
    wid = lax.axis_index("s") * nc + lax.axis_index("c")
    chunk = _B // 4
    j = wid // 4
    off = (wid % 4) * chunk
    base = wid * chunk
    pltpu.sync_copy(m_hbm.at[pl.ds(j, 1), pl.ds(off, chunk)], m_v)
    pltpu.sync_copy(gt_hbm, tbl_v)
    pltpu.sync_copy(s_hbm, s_v)
    pltpu.sync_copy(c_hbm, c_v)
    lanes = lax.iota(jnp.int32, _LANES)
    zeros = jnp.zeros((_LANES,), jnp.int32)
    for i in range(chunk // _LANES):
        idx = plsc.load_gather(m_v, [zeros, lanes + i * _LANES])
        cl = jnp.clip(idx, 0, g - 1)
        s = plsc.load_gather(s_v, [cl])
        c = plsc.load_gather(c_v, [cl])
        fg = idx >= 0
        lab = jnp.minimum(jnp.where(fg, 1.0, 0.0), s)
        lab = jnp.where(idx == -1, 0.0, lab)
        lab = jnp.where(idx == -2, -1.0, lab)
        lab = jnp.where(fg & (s < 1.0), -1.0, lab)
        lab = jnp.where(fg & (c == 0), -1.0, lab)
        ob = lanes * 5 + (i * _LANES * 5)
        plsc.store_scatter(o_v, [ob], lab)
        for k in range(4):
            bk = plsc.load_gather(tbl_v, [cl, jnp.full((_LANES,), k, jnp.int32)])
            plsc.store_scatter(o_v, [ob + (k + 1)], bk)
    tail = (n - (_NW - 1) * chunk) * 5
    @pl.when(wid < _NW - 1)
    def _():
        pltpu.sync_copy(o_v, out_hbm.at[pl.ds(base * 5, chunk * 5)])
    @pl.when(wid == _NW - 1)
    def _():
        pltpu.sync_copy(o_v.at[pl.ds(0, tail)],
                        out_hbm.at[pl.ds(base * 5, tail)])


def kernel(gt_boxes, anchors, score_labels, confidence_labels):
    n, g = anchors.shape[0], gt_boxes.shape[0]
    f32 = jnp.float32
    np_ = -(-n // _B) * _B
    nb = np_ // _B
    gp = -(-g // 8) * 8

    gt_all = jnp.zeros((gp, 8), f32).at[:g, 0:4].set(gt_boxes)

    matcher = pl.pallas_call(
        functools.partial(_matcher_body, n=n, gp=gp, g=g, nb=nb),
        out_shape=jax.ShapeDtypeStruct((nb, _B), jnp.int32),
        scratch_shapes=[
            pltpu.VMEM((nb, gp, _B), f32),
            pltpu.VMEM((gp, 128), f32),
        ],
    )
    matches = matcher(anchors, gt_all)

    sc_labels = functools.partial(
        pl.kernel,
        mesh=plsc.VectorSubcoreMesh(core_axis_name="c", subcore_axis_name="s"),
        compiler_params=pltpu.CompilerParams(needs_layout_passes=False),
        out_type=jax.ShapeDtypeStruct((np_ * 5,), f32),
        scratch_types=[
            pltpu.VMEM((1, _B // 4), jnp.int32),
            pltpu.VMEM((g, 4), f32),
            pltpu.VMEM((g,), f32),
            pltpu.VMEM((g,), jnp.int32),
            pltpu.VMEM((_B // 4 * 5,), f32),
        ],
    )(functools.partial(_sc_labels_body, n=n, g=g, nc=2))
    out_flat = sc_labels(matches, gt_boxes, score_labels, confidence_labels)
    return out_flat.reshape(np_, 5)[:n]


# speedup vs baseline: 1.2019x; 1.2019x over previous
import functools

import jax
import jax.numpy as jnp
from jax import lax
from jax.experimental import pallas as pl
from jax.experimental.pallas import tpu as pltpu
from jax.experimental.pallas import tpu_sc as plsc

LOW_THRESH = 0.3
HIGH_THRESH = 0.7

_NW = 32
_LANES = 16
_B = 2560


def _matcher_body(an_ref, gt_ref, out_ref, q_ref, gm_ref, *, n, gp, g, nb):
    gx1 = gt_ref[:, 0:1]
    gy1 = gt_ref[:, 1:2]
    gx2 = gt_ref[:, 2:3]
    gy2 = gt_ref[:, 3:4]
    ga = (gx2 - gx1) * (gy2 - gy1)
    widths = [min(_B, n - j * _B) for j in range(nb)]
    for j in range(nb):
        w_ = widths[j]
        sl = pl.ds(j * _B, w_)
        ax1 = an_ref[0:1, sl]
        ay1 = an_ref[1:2, sl]
        ax2 = an_ref[2:3, sl]
        ay2 = an_ref[3:4, sl]
        ab = (ax2 - ax1) * (ay2 - ay1)
        w = jnp.maximum(jnp.minimum(gx2, ax2) - jnp.maximum(gx1, ax1), 0.0)
        h = jnp.maximum(jnp.minimum(gy2, ay2) - jnp.maximum(gy1, ay1), 0.0)
        inter = w * h
        q = inter / (ga + ab - inter)
        q_ref[j, :, 0:w_] = q
        bm = jnp.max(q, axis=1, keepdims=True)
        if j == 0:
            gm_ref[:, 0:1] = bm
        else:
            gm_ref[:, 0:1] = jnp.maximum(gm_ref[:, 0:1], bm)
    gm = gm_ref[:, 0:1]
    for j in range(nb):
        w_ = widths[j]
        q = q_ref[j, :, 0:w_]
        giota = lax.broadcasted_iota(jnp.int32, (g, w_), 0)
        mv = jnp.max(q, axis=0, keepdims=True)
        am = jnp.min(jnp.where(q == mv, giota, g), axis=0, keepdims=True)
        restore = jnp.any(q == gm, axis=0, keepdims=True)
        m = jnp.where(mv < LOW_THRESH, -1, jnp.where(mv < HIGH_THRESH, -2, am))
        m = jnp.where(restore, am, m)
        out_ref[j:j + 1, 0:w_] = m


def _sc_labels_body(m_hbm, gt_hbm, s_hbm, c_hbm, out_hbm,
                    m_v, tbl_v, s_v, c_v, o_v, *, n, g, nc):
    wid = lax.axis_index("s") * nc + lax.axis_index("c")
    chunk = _B // 4
    j = wid // 4
    off = (wid % 4) * chunk
    base = wid * chunk
    pltpu.sync_copy(m_hbm.at[pl.ds(j, 1), pl.ds(off, chunk)], m_v)
    pltpu.sync_copy(gt_hbm, tbl_v)
    pltpu.sync_copy(s_hbm, s_v)
    pltpu.sync_copy(c_hbm, c_v)
    lanes = lax.iota(jnp.int32, _LANES)
    zeros = jnp.zeros((_LANES,), jnp.int32)
    for i in range(chunk // _LANES):
        idx = plsc.load_gather(m_v, [zeros, lanes + i * _LANES])
        cl = jnp.clip(idx, 0, g - 1)
        s = plsc.load_gather(s_v, [cl])
        c = plsc.load_gather(c_v, [cl])
        fg = idx >= 0
        lab = jnp.minimum(jnp.where(fg, 1.0, 0.0), s)
        lab = jnp.where(idx == -1, 0.0, lab)
        lab = jnp.where(idx == -2, -1.0, lab)
        lab = jnp.where(fg & (s < 1.0), -1.0, lab)
        lab = jnp.where(fg & (c == 0), -1.0, lab)
        ob = lanes * 5 + (i * _LANES * 5)
        plsc.store_scatter(o_v, [ob], lab)
        for k in range(4):
            bk = plsc.load_gather(tbl_v, [cl, jnp.full((_LANES,), k, jnp.int32)])
            plsc.store_scatter(o_v, [ob + (k + 1)], bk)
    tail = (n - (_NW - 1) * chunk) * 5
    @pl.when(wid < _NW - 1)
    def _():
        pltpu.sync_copy(o_v, out_hbm.at[pl.ds(base * 5, chunk * 5)])
    @pl.when(wid == _NW - 1)
    def _():
        pltpu.sync_copy(o_v.at[pl.ds(0, tail)],
                        out_hbm.at[pl.ds(base * 5, tail)])


def kernel(gt_boxes, anchors, score_labels, confidence_labels):
    n, g = anchors.shape[0], gt_boxes.shape[0]
    f32 = jnp.float32
    np_ = -(-n // _B) * _B
    nb = np_ // _B

    matcher = pl.pallas_call(
        functools.partial(_matcher_body, n=n, gp=g, g=g, nb=nb),
        out_shape=jax.ShapeDtypeStruct((nb, _B), jnp.int32),
        scratch_shapes=[
            pltpu.VMEM((nb, g, _B), f32),
            pltpu.VMEM((g, 128), f32),
        ],
    )
    matches = matcher(anchors.T, gt_boxes)

    sc_labels = functools.partial(
        pl.kernel,
        mesh=plsc.VectorSubcoreMesh(core_axis_name="c", subcore_axis_name="s"),
        compiler_params=pltpu.CompilerParams(needs_layout_passes=False),
        out_type=jax.ShapeDtypeStruct((np_ * 5,), f32),
        scratch_types=[
            pltpu.VMEM((1, _B // 4), jnp.int32),
            pltpu.VMEM((g, 4), f32),
            pltpu.VMEM((g,), f32),
            pltpu.VMEM((g,), jnp.int32),
            pltpu.VMEM((_B // 4 * 5,), f32),
        ],
    )(functools.partial(_sc_labels_body, n=n, g=g, nc=2))
    out_flat = sc_labels(matches, gt_boxes, score_labels, confidence_labels)
    return out_flat.reshape(np_, 5)[:n]


# speedup vs baseline: 1.5158x; 1.2612x over previous
import functools

import jax
import jax.numpy as jnp
from jax import lax
from jax.experimental import pallas as pl
from jax.experimental.pallas import tpu as pltpu
from jax.experimental.pallas import tpu_sc as plsc

LOW_THRESH = 0.3
HIGH_THRESH = 0.7

_NW = 32
_LANES = 16
_B = 2560


def _matcher_body(an_ref, gt_ref, out_ref, q_ref, gm_ref, *, n, gp, g, nb):
    gx1 = gt_ref[:, 0:1]
    gy1 = gt_ref[:, 1:2]
    gx2 = gt_ref[:, 2:3]
    gy2 = gt_ref[:, 3:4]
    ga = (gx2 - gx1) * (gy2 - gy1)
    widths = [min(_B, n - j * _B) for j in range(nb)]
    for j in range(nb):
        w_ = widths[j]
        sl = pl.ds(j * _B, w_)
        ax1 = an_ref[0:1, sl]
        ay1 = an_ref[1:2, sl]
        ax2 = an_ref[2:3, sl]
        ay2 = an_ref[3:4, sl]
        ab = (ax2 - ax1) * (ay2 - ay1)
        w = jnp.maximum(jnp.minimum(gx2, ax2) - jnp.maximum(gx1, ax1), 0.0)
        h = jnp.maximum(jnp.minimum(gy2, ay2) - jnp.maximum(gy1, ay1), 0.0)
        inter = w * h
        q = inter / (ga + ab - inter)
        q_ref[j, :, 0:w_] = q
        bm = jnp.max(q, axis=1, keepdims=True)
        if j == 0:
            gm_ref[:, 0:1] = bm
        else:
            gm_ref[:, 0:1] = jnp.maximum(gm_ref[:, 0:1], bm)
    gm = gm_ref[:, 0:1]
    for j in range(nb):
        w_ = widths[j]
        q = q_ref[j, :, 0:w_]
        giota = lax.broadcasted_iota(jnp.int32, (g, w_), 0)
        mv = jnp.max(q, axis=0, keepdims=True)
        am = jnp.min(jnp.where(q == mv, giota, g), axis=0, keepdims=True)
        restore = jnp.any(q == gm, axis=0, keepdims=True)
        m = jnp.where(mv < LOW_THRESH, -1, jnp.where(mv < HIGH_THRESH, -2, am))
        m = jnp.where(restore, am, m)
        out_ref[j:j + 1, 0:w_] = m


def _sc_labels_body(m_hbm, gt_hbm, s_hbm, c_hbm, out_hbm,
                    m_v, tbl_v, s_v, c_v, o_v, *, n, g, nc):
    wid = lax.axis_index("s") * nc + lax.axis_index("c")
    chunk = _B // 4
    j = wid // 4
    off = (wid % 4) * chunk
    base = wid * chunk
    pltpu.sync_copy(m_hbm.at[pl.ds(j, 1), pl.ds(off, chunk)], m_v)
    pltpu.sync_copy(gt_hbm, tbl_v)
    pltpu.sync_copy(s_hbm, s_v)
    pltpu.sync_copy(c_hbm, c_v)
    lanes = lax.iota(jnp.int32, _LANES)
    zeros = jnp.zeros((_LANES,), jnp.int32)
    for i in range(chunk // _LANES):
        idx = plsc.load_gather(m_v, [zeros, lanes + i * _LANES])
        cl = jnp.clip(idx, 0, g - 1)
        s = plsc.load_gather(s_v, [cl])
        c = plsc.load_gather(c_v, [cl])
        fg = idx >= 0
        lab = jnp.minimum(jnp.where(fg, 1.0, 0.0), s)
        lab = jnp.where(idx == -1, 0.0, lab)
        lab = jnp.where(idx == -2, -1.0, lab)
        lab = jnp.where(fg & (s < 1.0), -1.0, lab)
        lab = jnp.where(fg & (c == 0), -1.0, lab)
        rows = lanes + i * _LANES
        plsc.store_scatter(o_v, [rows, jnp.zeros((_LANES,), jnp.int32)], lab)
        for k in range(4):
            col = jnp.full((_LANES,), k, jnp.int32)
            bk = plsc.load_gather(tbl_v, [cl, col])
            plsc.store_scatter(o_v, [rows, col + 1], bk)
    tail = n - (_NW - 1) * chunk
    @pl.when(wid < _NW - 1)
    def _():
        pltpu.sync_copy(o_v, out_hbm.at[pl.ds(base, chunk), :])
    @pl.when(wid == _NW - 1)
    def _():
        pltpu.sync_copy(o_v.at[pl.ds(0, tail), :],
                        out_hbm.at[pl.ds(base, tail), :])


def kernel(gt_boxes, anchors, score_labels, confidence_labels):
    n, g = anchors.shape[0], gt_boxes.shape[0]
    f32 = jnp.float32
    np_ = -(-n // _B) * _B
    nb = np_ // _B

    matcher = pl.pallas_call(
        functools.partial(_matcher_body, n=n, gp=g, g=g, nb=nb),
        out_shape=jax.ShapeDtypeStruct((nb, _B), jnp.int32),
        scratch_shapes=[
            pltpu.VMEM((nb, g, _B), f32),
            pltpu.VMEM((g, 128), f32),
        ],
    )
    matches = matcher(anchors.T, gt_boxes)

    sc_labels = functools.partial(
        pl.kernel,
        mesh=plsc.VectorSubcoreMesh(core_axis_name="c", subcore_axis_name="s"),
        compiler_params=pltpu.CompilerParams(needs_layout_passes=False),
        out_type=jax.ShapeDtypeStruct((n, 5), f32),
        scratch_types=[
            pltpu.VMEM((1, _B // 4), jnp.int32),
            pltpu.VMEM((g, 4), f32),
            pltpu.VMEM((g,), f32),
            pltpu.VMEM((g,), jnp.int32),
            pltpu.VMEM((_B // 4, 5), f32),
        ],
    )(functools.partial(_sc_labels_body, n=n, g=g, nc=2))
    return sc_labels(matches, gt_boxes, score_labels, confidence_labels)


# speedup vs baseline: 1.5606x; 1.0295x over previous
import functools

import jax
import jax.numpy as jnp
from jax import lax
from jax.experimental import pallas as pl
from jax.experimental.pallas import tpu as pltpu
from jax.experimental.pallas import tpu_sc as plsc

LOW_THRESH = 0.3
HIGH_THRESH = 0.7

_NW = 32
_LANES = 16
_B = 2560


def _matcher_body(an_ref, gt_ref, out_ref, q_ref, gm_ref, *, n, gp, g, nb):
    gx1 = gt_ref[:, 0:1]
    gy1 = gt_ref[:, 1:2]
    gx2 = gt_ref[:, 2:3]
    gy2 = gt_ref[:, 3:4]
    ga = (gx2 - gx1) * (gy2 - gy1)
    widths = [min(_B, n - j * _B) for j in range(nb)]
    for j in range(nb):
        w_ = widths[j]
        sl = pl.ds(j * _B, w_)
        ax1 = an_ref[0:1, sl]
        ay1 = an_ref[1:2, sl]
        ax2 = an_ref[2:3, sl]
        ay2 = an_ref[3:4, sl]
        ab = (ax2 - ax1) * (ay2 - ay1)
        w = jnp.maximum(jnp.minimum(gx2, ax2) - jnp.maximum(gx1, ax1), 0.0)
        h = jnp.maximum(jnp.minimum(gy2, ay2) - jnp.maximum(gy1, ay1), 0.0)
        inter = w * h
        q = inter / (ga + ab - inter)
        q_ref[j, :, 0:w_] = q
        bm = jnp.max(q, axis=1, keepdims=True)
        if j == 0:
            gm_ref[:, 0:1] = bm
        else:
            gm_ref[:, 0:1] = jnp.maximum(gm_ref[:, 0:1], bm)
    gm = gm_ref[:, 0:1]
    for j in range(nb):
        w_ = widths[j]
        q = q_ref[j, :, 0:w_]
        giota = lax.broadcasted_iota(jnp.int32, (g, w_), 0)
        mv = jnp.max(q, axis=0, keepdims=True)
        am = jnp.min(jnp.where(q == mv, giota, g), axis=0, keepdims=True)
        restore = jnp.any(q == gm, axis=0, keepdims=True)
        m = jnp.where(mv < LOW_THRESH, -1, jnp.where(mv < HIGH_THRESH, -2, am))
        m = jnp.where(restore, am, m)
        out_ref[j:j + 1, 0:w_] = m


def _sc_labels_body(m_hbm, gt_hbm, s_hbm, c_hbm, out_hbm,
                    m_v, tbl_v, s_v, c_v, o_v, sem0, sem1, sem2, sem3,
                    *, n, g, nc):
    wid = lax.axis_index("s") * nc + lax.axis_index("c")
    chunk = _B // 4
    j = wid // 4
    off = (wid % 4) * chunk
    base = wid * chunk
    d0 = pltpu.async_copy(m_hbm.at[pl.ds(j, 1), pl.ds(off, chunk)], m_v, sem0)
    d1 = pltpu.async_copy(gt_hbm, tbl_v, sem1)
    d2 = pltpu.async_copy(s_hbm, s_v, sem2)
    d3 = pltpu.async_copy(c_hbm, c_v, sem3)
    d0.wait()
    d1.wait()
    d2.wait()
    d3.wait()
    lanes = lax.iota(jnp.int32, _LANES)
    zeros = jnp.zeros((_LANES,), jnp.int32)
    for i in range(chunk // _LANES):
        idx = plsc.load_gather(m_v, [zeros, lanes + i * _LANES])
        cl = jnp.clip(idx, 0, g - 1)
        s = plsc.load_gather(s_v, [cl])
        c = plsc.load_gather(c_v, [cl])
        fg = idx >= 0
        lab = jnp.minimum(jnp.where(fg, 1.0, 0.0), s)
        lab = jnp.where(idx == -1, 0.0, lab)
        lab = jnp.where(idx == -2, -1.0, lab)
        lab = jnp.where(fg & (s < 1.0), -1.0, lab)
        lab = jnp.where(fg & (c == 0), -1.0, lab)
        rows = lanes + i * _LANES
        plsc.store_scatter(o_v, [rows, jnp.zeros((_LANES,), jnp.int32)], lab)
        for k in range(4):
            col = jnp.full((_LANES,), k, jnp.int32)
            bk = plsc.load_gather(tbl_v, [cl, col])
            plsc.store_scatter(o_v, [rows, col + 1], bk)
    tail = n - (_NW - 1) * chunk
    @pl.when(wid < _NW - 1)
    def _():
        pltpu.sync_copy(o_v, out_hbm.at[pl.ds(base, chunk), :])
    @pl.when(wid == _NW - 1)
    def _():
        pltpu.sync_copy(o_v.at[pl.ds(0, tail), :],
                        out_hbm.at[pl.ds(base, tail), :])


def kernel(gt_boxes, anchors, score_labels, confidence_labels):
    n, g = anchors.shape[0], gt_boxes.shape[0]
    f32 = jnp.float32
    np_ = -(-n // _B) * _B
    nb = np_ // _B

    matcher = pl.pallas_call(
        functools.partial(_matcher_body, n=n, gp=g, g=g, nb=nb),
        out_shape=jax.ShapeDtypeStruct((nb, _B), jnp.int32),
        scratch_shapes=[
            pltpu.VMEM((nb, g, _B), f32),
            pltpu.VMEM((g, 128), f32),
        ],
    )
    matches = matcher(anchors.T, gt_boxes)

    sc_labels = functools.partial(
        pl.kernel,
        mesh=plsc.VectorSubcoreMesh(core_axis_name="c", subcore_axis_name="s"),
        compiler_params=pltpu.CompilerParams(needs_layout_passes=False),
        out_type=jax.ShapeDtypeStruct((n, 5), f32),
        scratch_types=[
            pltpu.VMEM((1, _B // 4), jnp.int32),
            pltpu.VMEM((g, 4), f32),
            pltpu.VMEM((g,), f32),
            pltpu.VMEM((g,), jnp.int32),
            pltpu.VMEM((_B // 4, 5), f32),
            pltpu.SemaphoreType.DMA,
            pltpu.SemaphoreType.DMA,
            pltpu.SemaphoreType.DMA,
            pltpu.SemaphoreType.DMA,
        ],
    )(functools.partial(_sc_labels_body, n=n, g=g, nc=2))
    return sc_labels(matches, gt_boxes, score_labels, confidence_labels)
